# Initial kernel scaffold; baseline (speedup 1.0000x reference)
import jax
import jax.numpy as jnp
from jax.experimental import pallas as pl


def kernel(inputs):
    raise NotImplementedError("write your pallas kernel here")


# speedup vs baseline: 1.1293x; 1.1293x over previous
import jax
import jax.numpy as jnp
from jax.experimental import pallas as pl
from jax.experimental.pallas import tpu as pltpu

_BS = 512


def _pad_kernel(in_ref, out_ref, min_ref):
    p = pl.program_id(0)
    b = pl.program_id(1)
    s = pl.program_id(2)

    @pl.when(p == 0)
    def _copy_and_reduce():
        x = in_ref[...]
        out_ref[...] = x
        m = jnp.min(x)
        first = (b == 0) & (s == 0)

        @pl.when(first)
        def _init():
            min_ref[0] = m

        @pl.when(jnp.logical_not(first))
        def _acc():
            min_ref[0] = jnp.minimum(min_ref[0], m)

    @pl.when(p == 1)
    def _fill_pad():
        out_ref[...] = jnp.full(out_ref.shape, min_ref[0] - 0.01,
                                out_ref.dtype)


def kernel(inputs):
    B, S, D = inputs.shape
    nb = S // _BS
    in_spec = pl.BlockSpec(
        (1, _BS, D),
        lambda p, b, s: (jnp.where(p == 0, b, B - 1),
                         jnp.where(p == 0, s, nb - 1), 0),
    )
    out_spec = pl.BlockSpec((1, _BS, D), lambda p, b, s: (b, p * nb + s, 0))
    return pl.pallas_call(
        _pad_kernel,
        grid=(2, B, nb),
        in_specs=[in_spec],
        out_specs=out_spec,
        out_shape=jax.ShapeDtypeStruct((B, 2 * S, D), inputs.dtype),
        scratch_shapes=[pltpu.SMEM((1,), jnp.float32)],
    )(inputs)


# speedup vs baseline: 1.3659x; 1.2095x over previous
import jax
import jax.numpy as jnp
from jax.experimental import pallas as pl
from jax.experimental.pallas import tpu as pltpu

_BS = 2048


def _pad_kernel(in_ref, out_ref, min_ref):
    p = pl.program_id(0)
    b = pl.program_id(1)
    s = pl.program_id(2)

    @pl.when(p == 0)
    def _copy_and_reduce():
        x = in_ref[...]
        out_ref[...] = x
        m = jnp.min(x)
        first = (b == 0) & (s == 0)

        @pl.when(first)
        def _init():
            min_ref[0] = m

        @pl.when(jnp.logical_not(first))
        def _acc():
            min_ref[0] = jnp.minimum(min_ref[0], m)

    @pl.when(p == 1)
    def _fill_pad():
        out_ref[...] = jnp.full(out_ref.shape, min_ref[0] - 0.01,
                                out_ref.dtype)


def kernel(inputs):
    B, S, D = inputs.shape
    nb = S // _BS
    in_spec = pl.BlockSpec(
        (1, _BS, D),
        lambda p, b, s: (jnp.where(p == 0, b, B - 1),
                         jnp.where(p == 0, s, nb - 1), 0),
    )
    out_spec = pl.BlockSpec((1, _BS, D), lambda p, b, s: (b, p * nb + s, 0))
    return pl.pallas_call(
        _pad_kernel,
        grid=(2, B, nb),
        in_specs=[in_spec],
        out_specs=out_spec,
        out_shape=jax.ShapeDtypeStruct((B, 2 * S, D), inputs.dtype),
        scratch_shapes=[pltpu.SMEM((1,), jnp.float32)],
    )(inputs)
